# baseline (speedup 1.0000x reference)
import jax
import jax.numpy as jnp
from jax import lax
from jax.experimental import pallas as pl
from jax.experimental.pallas import tpu as pltpu, tpu_sc as plsc

B = 16384
D = 32
CHUNK = 128
NROWS = B // CHUNK

_info = plsc.get_sparse_core_info()
_NC, _NS = _info.num_cores, _info.num_subcores
NW = _NC * _NS
ROWS_PER_W = NROWS // NW


def _gather_body(h_hbm, idx_hbm, out_hbm, idx_v, rows_v, sem):
    wid = lax.axis_index("s") * _NC + lax.axis_index("c")
    base = wid * ROWS_PER_W
    pltpu.sync_copy(idx_hbm.at[pl.ds(base, ROWS_PER_W)], idx_v)
    copies = [
        pltpu.async_copy(h_hbm.at[idx_v.at[j]], rows_v.at[j], sem)
        for j in range(ROWS_PER_W)
    ]
    for c in copies:
        c.wait()
    pltpu.sync_copy(rows_v, out_hbm.at[pl.ds(base, ROWS_PER_W)])


def kernel(h, idx):
    idx2d = idx.reshape(NROWS, CHUNK).astype(jnp.int32)
    gather = pl.kernel(
        _gather_body,
        out_type=jax.ShapeDtypeStruct((NROWS, CHUNK, D), jnp.float32),
        mesh=plsc.VectorSubcoreMesh(core_axis_name="c", subcore_axis_name="s"),
        scratch_types=[
            pltpu.VMEM((ROWS_PER_W, CHUNK), jnp.int32),
            pltpu.VMEM((ROWS_PER_W, CHUNK, D), jnp.float32),
            pltpu.SemaphoreType.DMA,
        ],
        compiler_params=pltpu.CompilerParams(use_tc_tiling_on_sc=False),
    )
    out = gather(h, idx2d)
    return out.reshape(1, B, D)


# speedup vs baseline: 3.7464x; 3.7464x over previous
import jax
import jax.numpy as jnp
from jax import lax
from jax.experimental import pallas as pl
from jax.experimental.pallas import tpu as pltpu, tpu_sc as plsc

B = 16384
D = 32
TL = 128
GRP = 16
NGRP = 32
RING = 24

_info = plsc.get_sparse_core_info()
_NC, _NS = _info.num_cores, _info.num_subcores
NW = _NC * _NS
PER_W = B // NW


def _gather_body(ht_hbm, idx_hbm, out_hbm, idx_v, slabs_v, buf_v, s0, s1):
    wid = lax.axis_index("s") * _NC + lax.axis_index("c")
    base = wid * PER_W
    pltpu.sync_copy(idx_hbm.at[pl.ds(base, PER_W)], idx_v)

    iota16 = lax.iota(jnp.int32, GRP)

    def issue_half(g, half, sem):
        gvec = idx_v[pl.ds(pl.multiple_of(g * GRP, GRP), GRP)]
        for t in range(8):
            i = g * GRP + half * 8 + t
            r = gvec[half * 8 + t]
            col = pl.multiple_of((r >> 7) << 7, TL)
            slot_lane = pl.multiple_of((i % RING) * TL, TL)
            pltpu.async_copy(
                ht_hbm.at[:, pl.ds(col, TL)],
                slabs_v.at[:, pl.ds(slot_lane, TL)],
                sem,
            )

    def wait_group(sem):
        pltpu.make_async_copy(
            ht_hbm.at[:, pl.ds(0, GRP * TL)],
            slabs_v.at[:, pl.ds(0, GRP * TL)],
            sem,
        ).wait()

    def extract_group(g):
        off = pl.multiple_of(g * GRP, GRP)
        d = idx_v[pl.ds(off, GRP)] & (TL - 1)
        lane_idx = ((iota16 + g * GRP) % RING) * TL + d
        for f in range(D):
            row = jnp.full((GRP,), f, jnp.int32)
            vals = plsc.load_gather(slabs_v, [row, lane_idx])
            buf_v[f, pl.ds(off, GRP)] = vals

    def step(g, sem_a, sem_b, issue2nd, issue1st):
        wait_group(sem_a)
        extract_group(g)
        if issue2nd:
            issue_half(g + 1, 1, sem_b)
        if issue1st:
            issue_half(g + 2, 0, sem_a)

    issue_half(0, 0, s0)
    issue_half(0, 1, s0)
    issue_half(1, 0, s1)

    def pair_body(gp, carry):
        a = gp * 2
        step(a, s0, s1, True, True)
        step(a + 1, s1, s0, True, True)
        return carry

    lax.fori_loop(0, (NGRP - 4) // 2, pair_body, 0)
    step(NGRP - 4, s0, s1, True, True)
    step(NGRP - 3, s1, s0, True, True)
    step(NGRP - 2, s0, s1, True, False)
    step(NGRP - 1, s1, s0, False, False)

    pltpu.sync_copy(buf_v, out_hbm.at[:, pl.ds(base, PER_W)])


def kernel(h, idx):
    ht = h.T
    idx32 = idx.astype(jnp.int32)
    gather = pl.kernel(
        _gather_body,
        out_type=jax.ShapeDtypeStruct((D, B), jnp.float32),
        mesh=plsc.VectorSubcoreMesh(core_axis_name="c", subcore_axis_name="s"),
        scratch_types=[
            pltpu.VMEM((PER_W,), jnp.int32),
            pltpu.VMEM((D, RING * TL), jnp.float32),
            pltpu.VMEM((D, PER_W), jnp.float32),
            pltpu.SemaphoreType.DMA,
            pltpu.SemaphoreType.DMA,
        ],
        compiler_params=pltpu.CompilerParams(
            use_tc_tiling_on_sc=True, needs_layout_passes=False),
    )
    out_t = gather(ht, idx32)
    return out_t.T.reshape(1, B, D)
